# baseline (speedup 1.0000x reference)
import functools

import jax
import jax.numpy as jnp
from jax import lax
from jax.experimental import pallas as pl
from jax.experimental.pallas import tpu as pltpu
from jax.experimental.pallas import tpu_sc as plsc

SEQ_LEN = 8192
FEAT_DIM = 2048
MAX_MISSING = 0.2

NUMEL = SEQ_LEN * FEAT_DIM
K_KEEP = int((1.0 - MAX_MISSING) * NUMEL)
RANK = NUMEL - K_KEEP

ROWS_PER_BLOCK = 512
NWORKERS = 32
ROWS_PER_W = SEQ_LEN // NWORKERS
CHUNK_ROWS = 8
NCHUNKS = ROWS_PER_W // CHUNK_ROWS
NBUF = 8


def _key_body(logits_ref, u_ref, out_ref):
    g = -jnp.log(-jnp.log(u_ref[...] + 1e-10))
    v = logits_ref[...] + g
    b = lax.bitcast_convert_type(v, jnp.int32)
    out_ref[...] = jnp.where(b >= 0, b, b ^ jnp.int32(0x7FFFFFFF))


_SC_MESH = plsc.VectorSubcoreMesh(core_axis_name="c", subcore_axis_name="s")
_SC_PARAMS = pltpu.CompilerParams(needs_layout_passes=False)

_HIST_SCRATCH = [
    pltpu.VMEM((CHUNK_ROWS, FEAT_DIM), jnp.int32),
    pltpu.VMEM((CHUNK_ROWS, FEAT_DIM), jnp.int32),
    pltpu.SemaphoreType.DMA,
    pltpu.SemaphoreType.DMA,
] + [pltpu.VMEM((16, 128), jnp.int32) for _ in range(NBUF)]


def _hist_pass(keys_hbm, out_hbm, chunk_a, chunk_b, sem_a, sem_b, hists,
               wid, digit16):
    base = wid * ROWS_PER_W
    zeros = jnp.zeros((16,), jnp.int32)
    ones = jnp.ones((16,), jnp.int32)

    def zrow(i, carry):
        for h in hists:
            for c in range(8):
                h[i, pl.ds(c * 16, 16)] = zeros
        return carry

    lax.fori_loop(0, 16, zrow, 0)

    def process(buf):
        def inner(i, c2):
            for r in range(CHUNK_ROWS):
                kv = buf[r, pl.ds(i * 16, 16)]
                d, m = digit16(kv)
                plsc.addupdate_scatter(hists[r % NBUF], [d >> 7, d & 127],
                                       ones, mask=m)
            return c2

        lax.fori_loop(0, FEAT_DIM // 16, inner, 0)

    def start(ci, buf, sem):
        pltpu.async_copy(
            keys_hbm.at[pl.ds(base + ci * CHUNK_ROWS, CHUNK_ROWS)], buf, sem)

    def drain(buf, sem):
        pltpu.make_async_copy(
            keys_hbm.at[pl.ds(base, CHUNK_ROWS)], buf, sem).wait()

    start(0, chunk_a, sem_a)

    def body2(j, carry):
        ci = 2 * j

        @pl.when(ci + 1 < NCHUNKS)
        def _():
            start(ci + 1, chunk_b, sem_b)

        drain(chunk_a, sem_a)
        process(chunk_a)

        @pl.when(ci + 2 < NCHUNKS)
        def _():
            start(ci + 2, chunk_a, sem_a)

        @pl.when(ci + 1 < NCHUNKS)
        def _():
            drain(chunk_b, sem_b)
            process(chunk_b)

        return carry

    lax.fori_loop(0, (NCHUNKS + 1) // 2, body2, 0)

    def mrow(i, carry):
        for c in range(8):
            sl = pl.ds(c * 16, 16)
            acc = hists[0][i, sl]
            for h in hists[1:]:
                acc = acc + h[i, sl]
            hists[0][i, sl] = acc
        return carry

    lax.fori_loop(0, 16, mrow, 0)
    pltpu.sync_copy(hists[0], out_hbm.at[wid])


_HIST_OUT = jax.ShapeDtypeStruct((NWORKERS, 16, 128), jnp.int32)


@functools.partial(pl.kernel, mesh=_SC_MESH, out_type=_HIST_OUT,
                   scratch_types=_HIST_SCRATCH, compiler_params=_SC_PARAMS)
def _hist_p1(keys_hbm, out_hbm, chunk_a, chunk_b, sem_a, sem_b, *hists):
    wid = lax.axis_index("s") * 2 + lax.axis_index("c")
    true16 = jnp.ones((16,), jnp.bool_)

    def digit16(kv):
        return (kv >> 21) + 1024, true16

    _hist_pass(keys_hbm, out_hbm, chunk_a, chunk_b, sem_a, sem_b, hists,
               wid, digit16)


@functools.partial(pl.kernel, mesh=_SC_MESH, out_type=_HIST_OUT,
                   scratch_types=_HIST_SCRATCH + [pltpu.VMEM((16,), jnp.int32)],
                   compiler_params=_SC_PARAMS)
def _hist_p2(keys_hbm, pvec_hbm, out_hbm, chunk_a, chunk_b, sem_a, sem_b,
             *hists_and_pbuf):
    *hists, pbuf = hists_and_pbuf
    wid = lax.axis_index("s") * 2 + lax.axis_index("c")
    pltpu.sync_copy(pvec_hbm, pbuf)
    pvec = pbuf[...]

    def digit16(kv):
        return (kv >> 10) & 0x7FF, ((kv >> 21) + 1024) == pvec

    _hist_pass(keys_hbm, out_hbm, chunk_a, chunk_b, sem_a, sem_b, hists,
               wid, digit16)


@functools.partial(pl.kernel, mesh=_SC_MESH, out_type=_HIST_OUT,
                   scratch_types=_HIST_SCRATCH + [pltpu.VMEM((16,), jnp.int32)],
                   compiler_params=_SC_PARAMS)
def _hist_p3(keys_hbm, pvec_hbm, out_hbm, chunk_a, chunk_b, sem_a, sem_b,
             *hists_and_pbuf):
    *hists, pbuf = hists_and_pbuf
    wid = lax.axis_index("s") * 2 + lax.axis_index("c")
    pltpu.sync_copy(pvec_hbm, pbuf)
    pvec = pbuf[...]

    def digit16(kv):
        return kv & 0x3FF, (kv >> 10) == pvec

    _hist_pass(keys_hbm, out_hbm, chunk_a, chunk_b, sem_a, sem_b, hists,
               wid, digit16)


def _cumsum_flat(h):
    hsum = jnp.sum(h, axis=0)
    i0 = lax.broadcasted_iota(jnp.int32, (128, 128), 0)
    i1 = lax.broadcasted_iota(jnp.int32, (128, 128), 1)
    upper = (i0 <= i1).astype(jnp.float32)
    row_cs = jnp.dot(hsum, upper, preferred_element_type=jnp.float32,
                     precision=lax.Precision.HIGHEST)
    j0 = lax.broadcasted_iota(jnp.int32, (16, 16), 0)
    j1 = lax.broadcasted_iota(jnp.int32, (16, 16), 1)
    strict_lower = (j1 < j0).astype(jnp.float32)
    row_tot = jnp.sum(hsum, axis=1, keepdims=True)
    prefix = jnp.dot(strict_lower, jnp.broadcast_to(row_tot, (16, 128)),
                     preferred_element_type=jnp.float32,
                     precision=lax.Precision.HIGHEST)
    return row_cs + prefix


def _digit_rank(hist_f32, rank_f32):
    cs = _cumsum_flat(hist_f32)
    le = cs <= rank_f32
    digit = jnp.sum(le.astype(jnp.float32)).astype(jnp.int32)
    below = jnp.max(jnp.where(le, cs, 0.0))
    return digit, rank_f32 - below


def _splat(ref, value):
    ref[...] = jnp.zeros((8, 128), jnp.int32) + value


def _select1_body(hist_ref, p_ref, rp_ref):
    d1, rp = _digit_rank(hist_ref[...].astype(jnp.float32), float(RANK))
    _splat(p_ref, d1)
    _splat(rp_ref, rp.astype(jnp.int32))


def _select2_body(hist_ref, p_smem, rp_smem, p_ref, rp_ref):
    d2, rp = _digit_rank(hist_ref[...].astype(jnp.float32),
                         rp_smem[0].astype(jnp.float32))
    p2 = ((p_smem[0] - 1024) << 11) | d2
    _splat(p_ref, p2)
    _splat(rp_ref, rp.astype(jnp.int32))


def _select3_body(hist_ref, p_smem, rp_smem, t_ref):
    d3, _ = _digit_rank(hist_ref[...].astype(jnp.float32),
                        rp_smem[0].astype(jnp.float32))
    _splat(t_ref, (p_smem[0] << 10) | d3)


def _mask_body(keys_ref, t_ref, out_ref):
    out_ref[...] = (keys_ref[...] >= t_ref[0]).astype(jnp.float32)


_SEL_IN = pl.BlockSpec((NWORKERS, 16, 128), lambda: (0, 0, 0))
_SEL_OUT = pl.BlockSpec((8, 128), lambda: (0, 0))
_SEL_SHAPE = jax.ShapeDtypeStruct((8, 128), jnp.int32)
_SMEM = pl.BlockSpec(memory_space=pltpu.SMEM)


def kernel(x, logits, u):
    del x
    grid = (SEQ_LEN // ROWS_PER_BLOCK,)
    bspec = pl.BlockSpec((ROWS_PER_BLOCK, FEAT_DIM), lambda i: (i, 0))

    keys2d = pl.pallas_call(
        _key_body,
        grid=grid,
        in_specs=[bspec, bspec],
        out_specs=bspec,
        out_shape=jax.ShapeDtypeStruct((SEQ_LEN, FEAT_DIM), jnp.int32),
    )(logits, u)

    hist1 = _hist_p1(keys2d)
    p1, rp1 = pl.pallas_call(
        _select1_body,
        in_specs=[_SEL_IN],
        out_specs=[_SEL_OUT] * 2,
        out_shape=[_SEL_SHAPE] * 2,
    )(hist1)

    hist2 = _hist_p2(keys2d, p1[0, 0:16])
    p2, rp2 = pl.pallas_call(
        _select2_body,
        in_specs=[_SEL_IN, _SMEM, _SMEM],
        out_specs=[_SEL_OUT] * 2,
        out_shape=[_SEL_SHAPE] * 2,
    )(hist2, p1[0, 0:1], rp1[0, 0:1])

    hist3 = _hist_p3(keys2d, p2[0, 0:16])
    tsplat = pl.pallas_call(
        _select3_body,
        in_specs=[_SEL_IN, _SMEM, _SMEM],
        out_specs=_SEL_OUT,
        out_shape=_SEL_SHAPE,
    )(hist3, p2[0, 0:1], rp2[0, 0:1])

    m_hard = pl.pallas_call(
        _mask_body,
        grid=grid,
        in_specs=[bspec, _SMEM],
        out_specs=bspec,
        out_shape=jax.ShapeDtypeStruct((SEQ_LEN, FEAT_DIM), jnp.float32),
    )(keys2d, tsplat[0, 0:1])
    return m_hard


# speedup vs baseline: 3.7945x; 3.7945x over previous
import functools

import jax
import jax.numpy as jnp
from jax import lax
from jax.experimental import pallas as pl
from jax.experimental.pallas import tpu as pltpu
from jax.experimental.pallas import tpu_sc as plsc

SEQ_LEN = 8192
FEAT_DIM = 2048
MAX_MISSING = 0.2

NUMEL = SEQ_LEN * FEAT_DIM
K_KEEP = int((1.0 - MAX_MISSING) * NUMEL)
RANK = NUMEL - K_KEEP

ROWS_PER_BLOCK = 512
HBINS = 65536
NWORKERS = 32
ROWS_PER_W = SEQ_LEN // NWORKERS
CHUNK_ROWS = 8
NCHUNKS = ROWS_PER_W // CHUNK_ROWS


def _key_body(logits_ref, u_ref, out_ref):
    g = -jnp.log(-jnp.log(u_ref[...] + 1e-10))
    v = logits_ref[...] + g
    b = lax.bitcast_convert_type(v, jnp.int32)
    out_ref[...] = jnp.where(b >= 0, b, b ^ jnp.int32(0x7FFFFFFF))


_SC_MESH = plsc.VectorSubcoreMesh(core_axis_name="c", subcore_axis_name="s")
_SC_PARAMS = pltpu.CompilerParams(needs_layout_passes=False)


def _zero_hist(hist_v):
    zeros = jnp.zeros((16,), jnp.int32)

    @plsc.parallel_loop(0, 512)
    def _(i):
        for c in range(8):
            hist_v[i, pl.ds(c * 16, 16)] = zeros


def _hist_pass(keys_hbm, out_hbm, chunk_a, chunk_b, sem_a, sem_b, hist_v,
               wid, scatter16):
    base = wid * ROWS_PER_W

    def process(buf):
        @plsc.parallel_loop(0, FEAT_DIM // 16, unroll=2)
        def _(i):
            for r in range(CHUNK_ROWS):
                kv = buf[r, pl.ds(i * 16, 16)]
                scatter16(kv)

    def start(ci, buf, sem):
        pltpu.async_copy(
            keys_hbm.at[pl.ds(base + ci * CHUNK_ROWS, CHUNK_ROWS)], buf, sem)

    def drain(buf, sem):
        pltpu.make_async_copy(
            keys_hbm.at[pl.ds(base, CHUNK_ROWS)], buf, sem).wait()

    start(0, chunk_a, sem_a)

    def body2(j, carry):
        ci = 2 * j

        @pl.when(ci + 1 < NCHUNKS)
        def _():
            start(ci + 1, chunk_b, sem_b)

        drain(chunk_a, sem_a)
        process(chunk_a)

        @pl.when(ci + 2 < NCHUNKS)
        def _():
            start(ci + 2, chunk_a, sem_a)

        @pl.when(ci + 1 < NCHUNKS)
        def _():
            drain(chunk_b, sem_b)
            process(chunk_b)

        return carry

    lax.fori_loop(0, (NCHUNKS + 1) // 2, body2, 0)
    pltpu.sync_copy(hist_v, out_hbm.at[wid])


_HIST_SCRATCH = [
    pltpu.VMEM((CHUNK_ROWS, FEAT_DIM), jnp.int32),
    pltpu.VMEM((CHUNK_ROWS, FEAT_DIM), jnp.int32),
    pltpu.SemaphoreType.DMA,
    pltpu.SemaphoreType.DMA,
    pltpu.VMEM((512, 128), jnp.int32),
]


@functools.partial(
    pl.kernel,
    mesh=_SC_MESH,
    out_type=jax.ShapeDtypeStruct((NWORKERS, 512, 128), jnp.int32),
    scratch_types=_HIST_SCRATCH,
    compiler_params=_SC_PARAMS,
)
def _hist_hi(keys_hbm, out_hbm, chunk_a, chunk_b, sem_a, sem_b, hist_v):
    wid = lax.axis_index("s") * 2 + lax.axis_index("c")
    _zero_hist(hist_v)
    ones = jnp.ones((16,), jnp.int32)

    def scatter16(kv):
        bucket = (kv >> 16) + 32768
        plsc.addupdate_scatter(hist_v, [bucket >> 7, bucket & 127], ones)

    _hist_pass(keys_hbm, out_hbm, chunk_a, chunk_b, sem_a, sem_b, hist_v,
               wid, scatter16)


@functools.partial(
    pl.kernel,
    mesh=_SC_MESH,
    out_type=jax.ShapeDtypeStruct((NWORKERS, 512, 128), jnp.int32),
    scratch_types=_HIST_SCRATCH + [pltpu.VMEM((16,), jnp.int32)],
    compiler_params=_SC_PARAMS,
)
def _hist_lo(keys_hbm, bvec_hbm, out_hbm, chunk_a, chunk_b, sem_a, sem_b,
             hist_v, bbuf_v):
    wid = lax.axis_index("s") * 2 + lax.axis_index("c")
    _zero_hist(hist_v)
    pltpu.sync_copy(bvec_hbm, bbuf_v)
    bvec = bbuf_v[...]
    ones = jnp.ones((16,), jnp.int32)

    def scatter16(kv):
        bucket = (kv >> 16) + 32768
        low = kv & 0xFFFF
        m = bucket == bvec
        plsc.addupdate_scatter(hist_v, [low >> 7, low & 127], ones, mask=m)

    _hist_pass(keys_hbm, out_hbm, chunk_a, chunk_b, sem_a, sem_b, hist_v,
               wid, scatter16)


def _cumsum_flat(h):
    hsum = jnp.sum(h, axis=0)
    i0 = lax.broadcasted_iota(jnp.int32, (128, 128), 0)
    i1 = lax.broadcasted_iota(jnp.int32, (128, 128), 1)
    upper = (i0 <= i1).astype(jnp.float32)
    row_cs = jnp.dot(hsum, upper, preferred_element_type=jnp.float32,
                     precision=lax.Precision.HIGHEST)
    j0 = lax.broadcasted_iota(jnp.int32, (512, 512), 0)
    j1 = lax.broadcasted_iota(jnp.int32, (512, 512), 1)
    strict_lower = (j1 < j0).astype(jnp.float32)
    row_tot = jnp.sum(hsum, axis=1, keepdims=True)
    prefix = jnp.dot(strict_lower, jnp.broadcast_to(row_tot, (512, 128)),
                     preferred_element_type=jnp.float32,
                     precision=lax.Precision.HIGHEST)
    return row_cs + prefix


def _select_hi_body(hist_ref, b_ref, rp_ref):
    cs = _cumsum_flat(hist_ref[...].astype(jnp.float32))
    le = cs <= float(RANK)
    bstar = jnp.sum(le.astype(jnp.float32)).astype(jnp.int32)
    below = jnp.max(jnp.where(le, cs, 0.0)).astype(jnp.int32)
    rp = RANK - below
    b_ref[...] = jnp.zeros((8, 128), jnp.int32) + bstar
    rp_ref[...] = jnp.zeros((8, 128), jnp.int32) + rp


def _select_lo_body(hist_ref, b_smem, rp_smem, t_ref):
    rp = rp_smem[0].astype(jnp.float32)
    cs = _cumsum_flat(hist_ref[...].astype(jnp.float32))
    lstar = jnp.sum((cs <= rp).astype(jnp.float32)).astype(jnp.int32)
    bstar = b_smem[0]
    t = ((bstar - 32768) << 16) | lstar
    t_ref[...] = jnp.zeros((8, 128), jnp.int32) + t


def _mask_body(keys_ref, t_ref, out_ref):
    out_ref[...] = (keys_ref[...] >= t_ref[0]).astype(jnp.float32)


def kernel(x, logits, u):
    del x
    grid = (SEQ_LEN // ROWS_PER_BLOCK,)
    bspec = pl.BlockSpec((ROWS_PER_BLOCK, FEAT_DIM), lambda i: (i, 0))

    keys2d = pl.pallas_call(
        _key_body,
        grid=grid,
        in_specs=[bspec, bspec],
        out_specs=bspec,
        out_shape=jax.ShapeDtypeStruct((SEQ_LEN, FEAT_DIM), jnp.int32),
    )(logits, u)

    hist1 = _hist_hi(keys2d)

    sel_b, sel_rp = pl.pallas_call(
        _select_hi_body,
        in_specs=[pl.BlockSpec((NWORKERS, 512, 128), lambda: (0, 0, 0))],
        out_specs=[pl.BlockSpec((8, 128), lambda: (0, 0))] * 2,
        out_shape=[jax.ShapeDtypeStruct((8, 128), jnp.int32)] * 2,
    )(hist1)

    bvec16 = sel_b[0, 0:16]
    hist2 = _hist_lo(keys2d, bvec16)

    tsplat = pl.pallas_call(
        _select_lo_body,
        in_specs=[
            pl.BlockSpec((NWORKERS, 512, 128), lambda: (0, 0, 0)),
            pl.BlockSpec(memory_space=pltpu.SMEM),
            pl.BlockSpec(memory_space=pltpu.SMEM),
        ],
        out_specs=pl.BlockSpec((8, 128), lambda: (0, 0)),
        out_shape=jax.ShapeDtypeStruct((8, 128), jnp.int32),
    )(hist2, sel_b[0, 0:1], sel_rp[0, 0:1])

    m_hard = pl.pallas_call(
        _mask_body,
        grid=grid,
        in_specs=[bspec, pl.BlockSpec(memory_space=pltpu.SMEM)],
        out_specs=bspec,
        out_shape=jax.ShapeDtypeStruct((SEQ_LEN, FEAT_DIM), jnp.float32),
    )(keys2d, tsplat[0, 0:1])
    return m_hard
